# baseline (speedup 1.0000x reference)
import jax
import jax.numpy as jnp
from jax import lax
from jax.experimental import pallas as pl
from jax.experimental.pallas import tpu as pltpu, tpu_sc as plsc

VOCAB = 1000000
EMBED_DIM = 64
PAD_DIM = 96
BATCH = 16384
FIELDS = 26

NC = 2
NS = 16
NW = NC * NS

CHUNK = 128
NCHUNK = BATCH // CHUNK
TASKS = FIELDS * NCHUNK
TASKS_PER_W = TASKS // NW
NBUF = 4
STEPS = TASKS_PER_W // NBUF


def _embed_kernel(idx_hbm, table_hbm, out_hbm,
                  ib, g0, g1, g2, g3, t0b, t1b, t2b, t3b, gsems, wsems):
    gs = [g0, g1, g2, g3]
    ts = [t0b, t1b, t2b, t3b]
    wid = lax.axis_index("s") * NC + lax.axis_index("c")
    t0 = wid * TASKS_PER_W

    def task_fc(k):
        t = t0 + k
        return t // NCHUNK, t % NCHUNK

    def stage_and_fire(k, b):
        f, c = task_fc(k)
        pltpu.sync_copy(idx_hbm.at[f, pl.ds(c * CHUNK, CHUNK)], ib.at[b])
        pltpu.async_copy(table_hbm.at[ib.at[b]], gs[b], gsems.at[b])

    for b in range(NBUF):
        stage_and_fire(b, b)

    iota = lax.broadcasted_iota(jnp.int32, (16,), 0)
    a_idx = [(d0 + iota) >> 3 for d0 in range(0, EMBED_DIM, 16)]
    s_idx = [(d0 + iota) & 7 for d0 in range(0, EMBED_DIM, 16)]

    def transpose_block(b):
        def grp(lg, carry):
            base = lg * 16
            for j in range(16):
                col = jnp.full((16,), 1, jnp.int32) * (base + j)
                for q in range(EMBED_DIM // 16):
                    x = gs[b][base + j, pl.ds(q * 16, 16)]
                    plsc.store_scatter(ts[b], [a_idx[q], s_idx[q], col], x)
            return carry
        lax.fori_loop(0, CHUNK // 16, grp, 0)

    def wb_dst(f, c):
        return out_hbm.at[f, :, c]

    def step(i, carry):
        for b in range(NBUF):
            k = i * NBUF + b
            f, c = task_fc(k)
            @pl.when(i > 0)
            def _():
                pltpu.make_async_copy(
                    ts[b].at[:, :, pl.ds(0, CHUNK)], wb_dst(f, c),
                    wsems.at[b]).wait()
            pltpu.make_async_copy(
                table_hbm.at[ib.at[b]], gs[b], gsems.at[b]).wait()
            transpose_block(b)
            pltpu.async_copy(ts[b].at[:, :, pl.ds(0, CHUNK)], wb_dst(f, c),
                             wsems.at[b])
            @pl.when(i < STEPS - 1)
            def _():
                stage_and_fire(k + NBUF, b)
        return carry

    lax.fori_loop(0, STEPS, step, 0)
    for b in range(NBUF):
        k = (STEPS - 1) * NBUF + b
        f, c = task_fc(k)
        pltpu.make_async_copy(
            ts[b].at[:, :, pl.ds(0, CHUNK)], wb_dst(f, c), wsems.at[b]).wait()


def kernel(embed_input, weight):
    idx_t = embed_input.T
    w_pad = jnp.pad(weight, ((0, 0), (0, PAD_DIM - EMBED_DIM)))
    mesh = plsc.VectorSubcoreMesh(core_axis_name="c", subcore_axis_name="s")
    o5 = pl.kernel(
        _embed_kernel,
        out_type=jax.ShapeDtypeStruct((FIELDS, 8, NCHUNK, 8, CHUNK),
                                      jnp.float32),
        mesh=mesh,
        compiler_params=pltpu.CompilerParams(use_tc_tiling_on_sc=False,
                                             needs_layout_passes=False),
        scratch_types=[
            pltpu.VMEM((NBUF, CHUNK), jnp.int32),
            pltpu.VMEM((CHUNK, PAD_DIM), jnp.float32),
            pltpu.VMEM((CHUNK, PAD_DIM), jnp.float32),
            pltpu.VMEM((CHUNK, PAD_DIM), jnp.float32),
            pltpu.VMEM((CHUNK, PAD_DIM), jnp.float32),
            pltpu.VMEM((8, 8, CHUNK + 1), jnp.float32),
            pltpu.VMEM((8, 8, CHUNK + 1), jnp.float32),
            pltpu.VMEM((8, 8, CHUNK + 1), jnp.float32),
            pltpu.VMEM((8, 8, CHUNK + 1), jnp.float32),
            pltpu.SemaphoreType.DMA((NBUF,)),
            pltpu.SemaphoreType.DMA((NBUF,)),
        ],
    )(idx_t, w_pad)
    return o5.transpose(2, 4, 0, 1, 3).reshape(BATCH, FIELDS, EMBED_DIM)
